# Initial kernel scaffold; baseline (speedup 1.0000x reference)
import jax
import jax.numpy as jnp
from jax.experimental import pallas as pl


def kernel(hidden_states, gate_w, gate_proj_w, up_proj_w, down_proj_w):
    raise NotImplementedError("write your pallas kernel here")


# speedup vs baseline: 5.6560x; 5.6560x over previous
import functools

import jax
import jax.numpy as jnp
from jax.experimental import pallas as pl
from jax.experimental.pallas import tpu as pltpu

_E = 8
_TOPK = 2


def _moe_body(x_ref, gate_w_ref, gp_ref, up_ref, dp_ref,
              out_ref, logits_ref, coef_ref):
    e = pl.program_id(0)
    M = x_ref.shape[0]

    @pl.when(e == 0)
    def _routing():
        x = x_ref[...]
        logits = jax.lax.dot_general(
            x, gate_w_ref[...],
            (((1,), (1,)), ((), ())),
            preferred_element_type=jnp.float32)
        logits_ref[...] = logits
        m = jnp.max(logits, axis=1, keepdims=True)
        p = jnp.exp(logits - m)
        p = p / jnp.sum(p, axis=1, keepdims=True)
        lane = jax.lax.broadcasted_iota(jnp.int32, p.shape, 1)
        e1 = jnp.argmax(p, axis=1)[:, None]
        mask1 = lane == e1
        p_masked = jnp.where(mask1, -jnp.inf, p)
        e2 = jnp.argmax(p_masked, axis=1)[:, None]
        mask2 = lane == e2
        m1 = jnp.max(p, axis=1, keepdims=True)
        m2 = jnp.max(p_masked, axis=1, keepdims=True)
        denom = m1 + m2
        coef_ref[...] = jnp.where(mask1 | mask2, p, 0.0) / denom

    x = x_ref[...]
    g = jax.lax.dot_general(
        x, gp_ref[0], (((1,), (1,)), ((), ())),
        preferred_element_type=jnp.float32)
    u = jax.lax.dot_general(
        x, up_ref[0], (((1,), (1,)), ((), ())),
        preferred_element_type=jnp.float32)
    h = (g * jax.nn.sigmoid(g)) * u
    y = jax.lax.dot_general(
        h, dp_ref[0], (((1,), (1,)), ((), ())),
        preferred_element_type=jnp.float32)
    c = coef_ref[...]
    lane = jax.lax.broadcasted_iota(jnp.int32, c.shape, 1)
    coef = jnp.sum(jnp.where(lane == e, c, 0.0), axis=1, keepdims=True)
    contrib = coef * y

    @pl.when(e == 0)
    def _init():
        out_ref[...] = contrib

    @pl.when(e != 0)
    def _acc():
        out_ref[...] += contrib


@functools.partial(jax.jit, static_argnames=())
def kernel(hidden_states, gate_w, gate_proj_w, up_proj_w, down_proj_w):
    B, S, H = hidden_states.shape
    M = B * S
    E, FF, _ = gate_proj_w.shape
    x = hidden_states.reshape(M, H)

    out, logits = pl.pallas_call(
        _moe_body,
        grid=(E,),
        in_specs=[
            pl.BlockSpec((M, H), lambda e: (0, 0)),
            pl.BlockSpec((E, H), lambda e: (0, 0)),
            pl.BlockSpec((1, FF, H), lambda e: (e, 0, 0)),
            pl.BlockSpec((1, FF, H), lambda e: (e, 0, 0)),
            pl.BlockSpec((1, H, FF), lambda e: (e, 0, 0)),
        ],
        out_specs=[
            pl.BlockSpec((M, H), lambda e: (0, 0)),
            pl.BlockSpec((M, E), lambda e: (0, 0)),
        ],
        out_shape=[
            jax.ShapeDtypeStruct((M, H), jnp.float32),
            jax.ShapeDtypeStruct((M, E), jnp.float32),
        ],
        scratch_shapes=[pltpu.VMEM((M, E), jnp.float32)],
        compiler_params=pltpu.CompilerParams(
            dimension_semantics=("arbitrary",),
        ),
    )(x, gate_w, gate_proj_w, up_proj_w, down_proj_w)

    return out.reshape(B, S, H), logits
